# baseline (speedup 1.0000x reference)
import functools

import jax
import jax.numpy as jnp
from jax import lax
from jax.experimental import pallas as pl
from jax.experimental.pallas import tpu as pltpu
from jax.experimental.pallas import tpu_sc as plsc

_B = 16384
_L = 200
_D = 32
_N = _B * _L

_NC = 2
_NS = 16
_NW = _NC * _NS

_SUB = 128
_K = 8
_CHUNK = _SUB * _K
_ROWS_PER_W = _N // _NW
_N_CHUNKS = _ROWS_PER_W // _CHUNK


@functools.cache
def _build():
    mesh = plsc.VectorSubcoreMesh(core_axis_name="c", subcore_axis_name="s")

    @functools.partial(
        pl.kernel,
        mesh=mesh,
        compiler_params=pltpu.CompilerParams(use_tc_tiling_on_sc=False),
        out_type=jax.ShapeDtypeStruct((_N, _D), jnp.float32),
        scratch_types=[
            pltpu.VMEM((_K, _SUB), jnp.int32),
            pltpu.VMEM((_CHUNK, _D), jnp.float32),
            pltpu.SemaphoreType.DMA,
        ],
    )
    def emb(idx_hbm, table_hbm, out_hbm, idx_v, rows_v, gsem):
        wid = lax.axis_index("s") * _NC + lax.axis_index("c")
        row0 = wid * _ROWS_PER_W

        def body(i, carry):
            cb = pl.multiple_of(row0 + i * _CHUNK, _CHUNK)
            pltpu.sync_copy(
                idx_hbm.at[pl.ds(pl.multiple_of(cb // _SUB, _K), _K)], idx_v)
            copies = [
                pltpu.async_copy(
                    table_hbm.at[idx_v.at[j]],
                    rows_v.at[pl.ds(j * _SUB, _SUB)],
                    gsem,
                )
                for j in range(_K)
            ]
            for c in copies:
                c.wait()
            pltpu.sync_copy(rows_v, out_hbm.at[pl.ds(cb, _CHUNK)])
            return carry

        lax.fori_loop(0, _N_CHUNKS, body, 0)

    return emb


def kernel(action, table):
    idx = action.reshape(_N // _SUB, _SUB)
    rows = _build()(idx, table)
    return rows.reshape(_B, _L * _D)


# speedup vs baseline: 1.0030x; 1.0030x over previous
import functools

import jax
import jax.numpy as jnp
from jax import lax
from jax.experimental import pallas as pl
from jax.experimental.pallas import tpu as pltpu
from jax.experimental.pallas import tpu_sc as plsc

_B = 16384
_L = 200
_D = 32
_N = _B * _L

_NC = 2
_NS = 16
_NW = _NC * _NS

_SUB = 128
_K = 8
_CHUNK = _SUB * _K
_ROWS_PER_W = _N // _NW
_N_CHUNKS = _ROWS_PER_W // _CHUNK


@functools.cache
def _build():
    mesh = plsc.VectorSubcoreMesh(core_axis_name="c", subcore_axis_name="s")

    @functools.partial(
        pl.kernel,
        mesh=mesh,
        compiler_params=pltpu.CompilerParams(use_tc_tiling_on_sc=False),
        out_type=jax.ShapeDtypeStruct((_N, _D), jnp.float32),
        scratch_types=[
            pltpu.VMEM((_K, _SUB), jnp.int32),
            pltpu.VMEM((_K, _SUB), jnp.int32),
            pltpu.VMEM((_CHUNK, _D), jnp.float32),
            pltpu.VMEM((_CHUNK, _D), jnp.float32),
            pltpu.SemaphoreType.DMA,
            pltpu.SemaphoreType.DMA,
            pltpu.SemaphoreType.DMA,
            pltpu.SemaphoreType.DMA,
            pltpu.SemaphoreType.DMA,
            pltpu.SemaphoreType.DMA,
        ],
    )
    def emb(idx_hbm, table_hbm, out_hbm, idx0, idx1, rows0, rows1,
            isem0, isem1, gsem0, gsem1, osem0, osem1):
        wid = lax.axis_index("s") * _NC + lax.axis_index("c")
        row0 = wid * _ROWS_PER_W

        def cbase(c):
            return pl.multiple_of(row0 + c * _CHUNK, _CHUNK)

        def idx_src(c):
            return idx_hbm.at[pl.ds(pl.multiple_of(cbase(c) // _SUB, _K), _K)]

        def out_dst(c):
            return out_hbm.at[pl.ds(cbase(c), _CHUNK)]

        def fire_gathers(idxb, rowsb, gsem):
            for j in range(_K):
                pltpu.async_copy(table_hbm.at[idxb.at[j]],
                                 rowsb.at[pl.ds(j * _SUB, _SUB)], gsem)

        def wait_gathers(idxb, rowsb, gsem):
            for j in range(_K):
                pltpu.make_async_copy(table_hbm.at[idxb.at[j]],
                                      rowsb.at[pl.ds(j * _SUB, _SUB)],
                                      gsem).wait()

        def wait_copy(src, dst, sem):
            pltpu.make_async_copy(src, dst, sem).wait()

        pltpu.async_copy(idx_src(0), idx0, isem0)
        pltpu.async_copy(idx_src(1), idx1, isem1)
        wait_copy(idx_src(0), idx0, isem0)
        fire_gathers(idx0, rows0, gsem0)

        def body(t, carry):
            c = t * 2
            wait_gathers(idx0, rows0, gsem0)
            pltpu.async_copy(rows0, out_dst(c), osem0)

            @pl.when(c + 2 < _N_CHUNKS)
            def _():
                pltpu.async_copy(idx_src(c + 2), idx0, isem0)

            wait_copy(idx_src(c + 1), idx1, isem1)

            @pl.when(t > 0)
            def _():
                wait_copy(rows1, out_dst(c - 1), osem1)

            fire_gathers(idx1, rows1, gsem1)

            wait_gathers(idx1, rows1, gsem1)
            pltpu.async_copy(rows1, out_dst(c + 1), osem1)

            @pl.when(c + 3 < _N_CHUNKS)
            def _():
                pltpu.async_copy(idx_src(c + 3), idx1, isem1)

            @pl.when(c + 2 < _N_CHUNKS)
            def _():
                wait_copy(idx_src(c + 2), idx0, isem0)
                wait_copy(rows0, out_dst(c), osem0)
                fire_gathers(idx0, rows0, gsem0)

            return carry

        lax.fori_loop(0, _N_CHUNKS // 2, body, 0)

        wait_copy(rows0, out_dst(_N_CHUNKS - 2), osem0)
        wait_copy(rows1, out_dst(_N_CHUNKS - 1), osem1)

    return emb


def kernel(action, table):
    idx = action.reshape(_N // _SUB, _SUB)
    rows = _build()(idx, table)
    return rows.reshape(_B, _L * _D)


# speedup vs baseline: 29.5686x; 29.4816x over previous
import functools

import jax
import jax.numpy as jnp
from jax import lax
from jax.experimental import pallas as pl
from jax.experimental.pallas import tpu as pltpu
from jax.experimental.pallas import tpu_sc as plsc

_B = 16384
_L = 200
_D = 32
_N = _B * _L

_NC = 2
_NS = 16
_NW = _NC * _NS

_SUB = 128
_K = 8
_CHUNK = _SUB * _K
_ROWS_PER_W = _N // _NW
_N_CHUNKS = _ROWS_PER_W // _CHUNK


@functools.cache
def _build():
    mesh = plsc.VectorSubcoreMesh(core_axis_name="c", subcore_axis_name="s")

    @functools.partial(
        pl.kernel,
        mesh=mesh,
        compiler_params=pltpu.CompilerParams(use_tc_tiling_on_sc=False),
        out_type=jax.ShapeDtypeStruct((_N, _D), jnp.float32),
        scratch_types=[
            pltpu.VMEM((_K, _SUB), jnp.int32),
            pltpu.VMEM((_K, _SUB), jnp.int32),
            pltpu.VMEM((_CHUNK, _D), jnp.float32),
            pltpu.VMEM((_CHUNK, _D), jnp.float32),
            pltpu.VMEM_SHARED((6, _D), jnp.float32),
            pltpu.SemaphoreType.DMA,
            pltpu.SemaphoreType.DMA,
            pltpu.SemaphoreType.DMA,
            pltpu.SemaphoreType.DMA,
            pltpu.SemaphoreType.DMA,
            pltpu.SemaphoreType.DMA,
        ],
    )
    def emb(idx_hbm, table_hbm, out_hbm, idx0, idx1, rows0, rows1, table_v,
            isem0, isem1, gsem0, gsem1, osem0, osem1):
        wid = lax.axis_index("s") * _NC + lax.axis_index("c")
        row0 = wid * _ROWS_PER_W

        @pl.when(lax.axis_index("s") == 0)
        def _():
            pltpu.sync_copy(table_hbm, table_v)
        plsc.subcore_barrier()

        def cbase(c):
            return pl.multiple_of(row0 + c * _CHUNK, _CHUNK)

        def idx_src(c):
            return idx_hbm.at[pl.ds(pl.multiple_of(cbase(c) // _SUB, _K), _K)]

        def out_dst(c):
            return out_hbm.at[pl.ds(cbase(c), _CHUNK)]

        def fire_gathers(idxb, rowsb, gsem):
            for j in range(_K):
                pltpu.async_copy(table_v.at[idxb.at[j]],
                                 rowsb.at[pl.ds(j * _SUB, _SUB)], gsem)

        def wait_gathers(idxb, rowsb, gsem):
            for j in range(_K):
                pltpu.make_async_copy(table_v.at[idxb.at[j]],
                                      rowsb.at[pl.ds(j * _SUB, _SUB)],
                                      gsem).wait()

        def wait_copy(src, dst, sem):
            pltpu.make_async_copy(src, dst, sem).wait()

        pltpu.async_copy(idx_src(0), idx0, isem0)
        pltpu.async_copy(idx_src(1), idx1, isem1)
        wait_copy(idx_src(0), idx0, isem0)
        fire_gathers(idx0, rows0, gsem0)

        def body(t, carry):
            c = t * 2
            wait_gathers(idx0, rows0, gsem0)
            pltpu.async_copy(rows0, out_dst(c), osem0)

            @pl.when(c + 2 < _N_CHUNKS)
            def _():
                pltpu.async_copy(idx_src(c + 2), idx0, isem0)

            wait_copy(idx_src(c + 1), idx1, isem1)

            @pl.when(t > 0)
            def _():
                wait_copy(rows1, out_dst(c - 1), osem1)

            fire_gathers(idx1, rows1, gsem1)

            wait_gathers(idx1, rows1, gsem1)
            pltpu.async_copy(rows1, out_dst(c + 1), osem1)

            @pl.when(c + 3 < _N_CHUNKS)
            def _():
                pltpu.async_copy(idx_src(c + 3), idx1, isem1)

            @pl.when(c + 2 < _N_CHUNKS)
            def _():
                wait_copy(idx_src(c + 2), idx0, isem0)
                wait_copy(rows0, out_dst(c), osem0)
                fire_gathers(idx0, rows0, gsem0)

            return carry

        lax.fori_loop(0, _N_CHUNKS // 2, body, 0)

        wait_copy(rows0, out_dst(_N_CHUNKS - 2), osem0)
        wait_copy(rows1, out_dst(_N_CHUNKS - 1), osem1)

    return emb


def kernel(action, table):
    idx = action.reshape(_N // _SUB, _SUB)
    rows = _build()(idx, table)
    return rows.reshape(_B, _L * _D)


# speedup vs baseline: 35.3747x; 1.1964x over previous
import functools

import jax
import jax.numpy as jnp
from jax import lax
from jax.experimental import pallas as pl
from jax.experimental.pallas import tpu as pltpu
from jax.experimental.pallas import tpu_sc as plsc

_B = 16384
_L = 200
_D = 32
_N = _B * _L

_NC = 2
_NS = 16
_NW = _NC * _NS

_F = 4
_CD = _F * _D
_NCOMP = 6 ** _F
_N4 = _N // _F

_SUB = 128
_K = 2
_CSUB = _SUB * _K
_CHUNK = _CSUB * _F
_ROWS_PER_W = _N4 // _NW
_N_CHUNKS = _ROWS_PER_W // _CSUB


@functools.cache
def _build():
    mesh = plsc.VectorSubcoreMesh(core_axis_name="c", subcore_axis_name="s")

    @functools.partial(
        pl.kernel,
        mesh=mesh,
        compiler_params=pltpu.CompilerParams(use_tc_tiling_on_sc=False,
                                             needs_layout_passes=False),
        out_type=jax.ShapeDtypeStruct((_N4, _CD), jnp.float32),
        scratch_types=[
            pltpu.VMEM((_CHUNK,), jnp.int32),
            pltpu.VMEM((_CHUNK,), jnp.int32),
            pltpu.VMEM((_CSUB,), jnp.int32),
            pltpu.VMEM((_CSUB,), jnp.int32),
            pltpu.VMEM((_CSUB, _CD), jnp.float32),
            pltpu.VMEM((_CSUB, _CD), jnp.float32),
            pltpu.VMEM_SHARED((_NCOMP, _CD), jnp.float32),
            pltpu.SemaphoreType.DMA,
            pltpu.SemaphoreType.DMA,
            pltpu.SemaphoreType.DMA,
            pltpu.SemaphoreType.DMA,
            pltpu.SemaphoreType.DMA,
            pltpu.SemaphoreType.DMA,
        ],
    )
    def emb(idx_hbm, ctable_hbm, out_hbm, idx0, idx1, cidx0, cidx1,
            rows0, rows1, ct_v, isem0, isem1, gsem0, gsem1, osem0, osem1):
        wid = lax.axis_index("s") * _NC + lax.axis_index("c")
        crow0 = wid * _ROWS_PER_W

        @pl.when(lax.axis_index("s") == 0)
        def _():
            pltpu.sync_copy(ctable_hbm, ct_v)
        plsc.subcore_barrier()

        def cbase(c):
            return pl.multiple_of(crow0 + c * _CSUB, _CSUB)

        def idx_src(c):
            return idx_hbm.at[pl.ds(pl.multiple_of(cbase(c) * _F, _CHUNK),
                                    _CHUNK)]

        def out_dst(c):
            return out_hbm.at[pl.ds(cbase(c), _CSUB)]

        lanes = lax.iota(jnp.int32, 16)

        def compute_comp(idxb, cidxb):
            for g in range(_CSUB // 16):
                base = lanes * _F + g * 64
                a0 = plsc.load_gather(idxb, [base])
                a1 = plsc.load_gather(idxb, [base + 1])
                a2 = plsc.load_gather(idxb, [base + 2])
                a3 = plsc.load_gather(idxb, [base + 3])
                cidxb[pl.ds(g * 16, 16)] = ((a0 * 6 + a1) * 6 + a2) * 6 + a3

        def fire_gathers(cidxb, rowsb, gsem):
            for j in range(_K):
                pltpu.async_copy(ct_v.at[cidxb.at[pl.ds(j * _SUB, _SUB)]],
                                 rowsb.at[pl.ds(j * _SUB, _SUB)], gsem)

        def wait_gathers(cidxb, rowsb, gsem):
            for j in range(_K):
                pltpu.make_async_copy(ct_v.at[cidxb.at[pl.ds(j * _SUB, _SUB)]],
                                      rowsb.at[pl.ds(j * _SUB, _SUB)],
                                      gsem).wait()

        def wait_copy(src, dst, sem):
            pltpu.make_async_copy(src, dst, sem).wait()

        pltpu.async_copy(idx_src(0), idx0, isem0)
        pltpu.async_copy(idx_src(1), idx1, isem1)
        wait_copy(idx_src(0), idx0, isem0)
        compute_comp(idx0, cidx0)
        fire_gathers(cidx0, rows0, gsem0)

        def body(t, carry):
            c = t * 2
            wait_gathers(cidx0, rows0, gsem0)
            pltpu.async_copy(rows0, out_dst(c), osem0)

            @pl.when(c + 2 < _N_CHUNKS)
            def _():
                pltpu.async_copy(idx_src(c + 2), idx0, isem0)

            wait_copy(idx_src(c + 1), idx1, isem1)
            compute_comp(idx1, cidx1)

            @pl.when(t > 0)
            def _():
                wait_copy(rows1, out_dst(c - 1), osem1)

            fire_gathers(cidx1, rows1, gsem1)

            wait_gathers(cidx1, rows1, gsem1)
            pltpu.async_copy(rows1, out_dst(c + 1), osem1)

            @pl.when(c + 3 < _N_CHUNKS)
            def _():
                pltpu.async_copy(idx_src(c + 3), idx1, isem1)

            @pl.when(c + 2 < _N_CHUNKS)
            def _():
                wait_copy(idx_src(c + 2), idx0, isem0)
                compute_comp(idx0, cidx0)
                wait_copy(rows0, out_dst(c), osem0)
                fire_gathers(cidx0, rows0, gsem0)

            return carry

        lax.fori_loop(0, _N_CHUNKS // 2, body, 0)

        wait_copy(rows0, out_dst(_N_CHUNKS - 2), osem0)
        wait_copy(rows1, out_dst(_N_CHUNKS - 1), osem1)

    return emb


def kernel(action, table):
    idx = action.reshape(_N)
    combos = jnp.arange(_NCOMP)
    ctable = jnp.concatenate(
        [jnp.take(table, (combos // (6 ** (_F - 1 - m))) % 6, axis=0)
         for m in range(_F)], axis=1)
    rows = _build()(idx, ctable)
    return rows.reshape(_B, _L * _D)
